# baseline (speedup 1.0000x reference)
import functools

import jax
import jax.numpy as jnp
from jax import lax
from jax.experimental import pallas as pl
from jax.experimental.pallas import tpu as pltpu
from jax.experimental.pallas import tpu_sc as plsc

D = 128
L = 50
EPS = 1e-12
NW = 32
CHUNK = 80
BBC = 512
S = 2
LS = L // S


@functools.lru_cache(maxsize=None)
def _make_sc_gather(n_chunks: int):
    tok = NW * n_chunks * CHUNK
    per_w = n_chunks * CHUNK
    mesh = plsc.VectorSubcoreMesh(core_axis_name="c", subcore_axis_name="s")

    @functools.partial(
        pl.kernel, mesh=mesh,
        out_type=jax.ShapeDtypeStruct((tok, D), jnp.float32),
        scratch_types=[
            pltpu.VMEM((n_chunks, CHUNK), jnp.int32),
            pltpu.VMEM((CHUNK, D), jnp.float32),
            pltpu.VMEM((CHUNK, D), jnp.float32),
            pltpu.SemaphoreType.DMA,
            pltpu.SemaphoreType.DMA,
        ],
    )
    def sc_gather(ids_hbm, table_hbm, out_hbm, idx_v, buf0, buf1, sem0, sem1):
        wid = lax.axis_index("s") * 2 + lax.axis_index("c")
        base = wid * per_w
        pltpu.sync_copy(ids_hbm.at[wid], idx_v)
        pltpu.async_copy(table_hbm.at[idx_v.at[0]], buf0, sem0)
        n2 = n_chunks // 2

        def pair(jj, carry):
            j0 = 2 * jj
            pltpu.async_copy(table_hbm.at[idx_v.at[j0 + 1]], buf1, sem1)
            pltpu.make_async_copy(table_hbm.at[idx_v.at[j0]], buf0, sem0).wait()
            pltpu.sync_copy(buf0, out_hbm.at[pl.ds(base + j0 * CHUNK, CHUNK)])

            @pl.when(jj < n2 - 1)
            def _():
                pltpu.async_copy(table_hbm.at[idx_v.at[j0 + 2]], buf0, sem0)

            pltpu.make_async_copy(table_hbm.at[idx_v.at[j0 + 1]], buf1, sem1).wait()
            pltpu.sync_copy(buf1, out_hbm.at[pl.ds(base + (j0 + 1) * CHUNK, CHUNK)])
            return carry

        lax.fori_loop(0, n2, pair, 0)

    return sc_gather


def _tc_body_first(g, ttf, ctx, pos, w5, bias, gam, bet, out):
    _tc_compute(g, ttf, ctx, pos, w5, bias, gam, bet, out)


def _tc_body_acc(prev, g, ttf, ctx, pos, w5, bias, gam, bet, out):
    del prev
    _tc_compute(g, ttf, ctx, pos, w5, bias, gam, bet, out)


def _tc_compute(g, ttf, ctx, pos, w5, bias, gam, bet, out):
    x = g[...] + pos[...][:, None, :]
    c5 = jnp.concatenate([ctx[...], ttf[0][:, None, :]], axis=1)
    w5b = jnp.broadcast_to(w5[...][None], (LS, D, 5))
    projT = lax.dot_general(w5b, c5, (((2,), (1,)), ((0,), (0,))),
                            preferred_element_type=jnp.float32)
    proj = jnp.swapaxes(projT, 1, 2)
    x = x + proj + bias[0][None, None, :]
    mu = jnp.mean(x, axis=-1, keepdims=True)
    xc = x - mu
    var = jnp.mean(xc * xc, axis=-1, keepdims=True)
    y = xc * lax.rsqrt(var + EPS)
    out[...] = y * gam[0][None, None, :] + bet[0][None, None, :]


def kernel(input_ids, token_type_ids, context_feature, word_emb, pos_emb,
           type_emb, ctx_W, ctx_b, gamma, beta):
    B, Lx = input_ids.shape
    assert Lx == L and (LS * B) % (NW * CHUNK) == 0
    n_chunks = (LS * B) // (NW * CHUNK)

    ids4 = input_ids.astype(jnp.int32).T.reshape(S, NW, n_chunks, CHUNK)
    ttf4 = token_type_ids.T.astype(jnp.float32).reshape(S, LS, B)
    ctx3 = jnp.transpose(context_feature, (1, 2, 0))
    pos_s = [pos_emb[s * LS:(s + 1) * LS] for s in range(S)]
    w5T = jnp.concatenate([ctx_W, (type_emb[1] - type_emb[0])[None, :]],
                          axis=0).T
    bias = (ctx_b + type_emb[0]).reshape(1, D)
    gam2 = gamma.reshape(1, D)
    bet2 = beta.reshape(1, D)

    sc = _make_sc_gather(n_chunks)
    g3s = []
    for s in range(S):
        g2 = sc(ids4[s], word_emb)
        g3s.append(g2.reshape(LS, B, D))

    outT = None
    for s in range(S):
        data_specs = [
            pl.BlockSpec((LS, BBC, D), lambda i: (0, i, 0)),
            pl.BlockSpec((1, LS, BBC), lambda i, s=s: (s, 0, i)),
            pl.BlockSpec((LS, 4, BBC), lambda i, s=s: (s, 0, i)),
            pl.BlockSpec((LS, D), lambda i: (0, 0)),
            pl.BlockSpec((D, 5), lambda i: (0, 0)),
            pl.BlockSpec((1, D), lambda i: (0, 0)),
            pl.BlockSpec((1, D), lambda i: (0, 0)),
            pl.BlockSpec((1, D), lambda i: (0, 0)),
        ]
        out_spec = pl.BlockSpec((LS, BBC, D), lambda i, s=s: (s, i, 0))
        args = (g3s[s], ttf4, ctx3, pos_s[s], w5T, bias, gam2, bet2)
        if s == 0:
            outT = pl.pallas_call(
                _tc_body_first,
                out_shape=jax.ShapeDtypeStruct((L, B, D), jnp.float32),
                grid=(B // BBC,),
                in_specs=data_specs,
                out_specs=out_spec,
            )(*args)
        else:
            outT = pl.pallas_call(
                _tc_body_acc,
                out_shape=jax.ShapeDtypeStruct((L, B, D), jnp.float32),
                grid=(B // BBC,),
                in_specs=[pl.BlockSpec(memory_space=pl.ANY)] + data_specs,
                out_specs=out_spec,
                input_output_aliases={0: 0},
            )(outT, *args)
    return jnp.transpose(outT, (1, 0, 2))


# speedup vs baseline: 1.0134x; 1.0134x over previous
import functools

import jax
import jax.numpy as jnp
from jax import lax
from jax.experimental import pallas as pl
from jax.experimental.pallas import tpu as pltpu
from jax.experimental.pallas import tpu_sc as plsc

D = 128
L = 50
EPS = 1e-12
NW = 32
CHUNK = 128
BBC = 1024
S = 5
LS = L // S


@functools.lru_cache(maxsize=None)
def _make_sc_gather(n_chunks: int):
    tok = NW * n_chunks * CHUNK
    per_w = n_chunks * CHUNK
    mesh = plsc.VectorSubcoreMesh(core_axis_name="c", subcore_axis_name="s")

    @functools.partial(
        pl.kernel, mesh=mesh,
        out_type=jax.ShapeDtypeStruct((tok, D), jnp.float32),
        scratch_types=[
            pltpu.VMEM((n_chunks, CHUNK), jnp.int32),
            pltpu.VMEM((CHUNK, D), jnp.float32),
            pltpu.VMEM((CHUNK, D), jnp.float32),
            pltpu.SemaphoreType.DMA,
            pltpu.SemaphoreType.DMA,
        ],
    )
    def sc_gather(ids_hbm, table_hbm, out_hbm, idx_v, buf0, buf1, sem0, sem1):
        wid = lax.axis_index("s") * 2 + lax.axis_index("c")
        base = wid * per_w
        pltpu.sync_copy(ids_hbm.at[wid], idx_v)
        pltpu.async_copy(table_hbm.at[idx_v.at[0]], buf0, sem0)
        n2 = n_chunks // 2

        def pair(jj, carry):
            j0 = 2 * jj
            pltpu.async_copy(table_hbm.at[idx_v.at[j0 + 1]], buf1, sem1)
            pltpu.make_async_copy(table_hbm.at[idx_v.at[j0]], buf0, sem0).wait()
            pltpu.sync_copy(buf0, out_hbm.at[pl.ds(base + j0 * CHUNK, CHUNK)])

            @pl.when(jj < n2 - 1)
            def _():
                pltpu.async_copy(table_hbm.at[idx_v.at[j0 + 2]], buf0, sem0)

            pltpu.make_async_copy(table_hbm.at[idx_v.at[j0 + 1]], buf1, sem1).wait()
            pltpu.sync_copy(buf1, out_hbm.at[pl.ds(base + (j0 + 1) * CHUNK, CHUNK)])
            return carry

        lax.fori_loop(0, n2, pair, 0)

    return sc_gather


def _tc_body_first(g, ttf, ctx, pos, w5, bias, gam, bet, out):
    _tc_compute(g, ttf, ctx, pos, w5, bias, gam, bet, out)


def _tc_body_acc(prev, g, ttf, ctx, pos, w5, bias, gam, bet, out):
    del prev
    _tc_compute(g, ttf, ctx, pos, w5, bias, gam, bet, out)


def _tc_compute(g, ttf, ctx, pos, w5, bias, gam, bet, out):
    x = g[...] + pos[...][:, None, :]
    c5 = jnp.concatenate([ctx[...], ttf[0][:, None, :]], axis=1)
    w5b = jnp.broadcast_to(w5[...][None], (LS, D, 5))
    projT = lax.dot_general(w5b, c5, (((2,), (1,)), ((0,), (0,))),
                            preferred_element_type=jnp.float32)
    proj = jnp.swapaxes(projT, 1, 2)
    x = x + proj + bias[0][None, None, :]
    mu = jnp.mean(x, axis=-1, keepdims=True)
    xc = x - mu
    var = jnp.mean(xc * xc, axis=-1, keepdims=True)
    y = xc * lax.rsqrt(var + EPS)
    out[...] = y * gam[0][None, None, :] + bet[0][None, None, :]


def kernel(input_ids, token_type_ids, context_feature, word_emb, pos_emb,
           type_emb, ctx_W, ctx_b, gamma, beta):
    B, Lx = input_ids.shape
    assert Lx == L and (LS * B) % (NW * CHUNK) == 0
    n_chunks = (LS * B) // (NW * CHUNK)

    ids4 = input_ids.astype(jnp.int32).T.reshape(S, NW, n_chunks, CHUNK)
    ttf4 = token_type_ids.T.astype(jnp.float32).reshape(S, LS, B)
    ctx3 = jnp.transpose(context_feature, (1, 2, 0))
    pos_s = [pos_emb[s * LS:(s + 1) * LS] for s in range(S)]
    w5T = jnp.concatenate([ctx_W, (type_emb[1] - type_emb[0])[None, :]],
                          axis=0).T
    bias = (ctx_b + type_emb[0]).reshape(1, D)
    gam2 = gamma.reshape(1, D)
    bet2 = beta.reshape(1, D)

    sc = _make_sc_gather(n_chunks)
    g3s = []
    for s in range(S):
        g2 = sc(ids4[s], word_emb)
        g3s.append(g2.reshape(LS, B, D))

    outT = None
    for s in range(S):
        data_specs = [
            pl.BlockSpec((LS, BBC, D), lambda i: (0, i, 0)),
            pl.BlockSpec((1, LS, BBC), lambda i, s=s: (s, 0, i)),
            pl.BlockSpec((LS, 4, BBC), lambda i, s=s: (s, 0, i)),
            pl.BlockSpec((LS, D), lambda i: (0, 0)),
            pl.BlockSpec((D, 5), lambda i: (0, 0)),
            pl.BlockSpec((1, D), lambda i: (0, 0)),
            pl.BlockSpec((1, D), lambda i: (0, 0)),
            pl.BlockSpec((1, D), lambda i: (0, 0)),
        ]
        out_spec = pl.BlockSpec((LS, BBC, D), lambda i, s=s: (s, i, 0))
        args = (g3s[s], ttf4, ctx3, pos_s[s], w5T, bias, gam2, bet2)
        if s == 0:
            outT = pl.pallas_call(
                _tc_body_first,
                out_shape=jax.ShapeDtypeStruct((L, B, D), jnp.float32),
                grid=(B // BBC,),
                in_specs=data_specs,
                out_specs=out_spec,
            )(*args)
        else:
            outT = pl.pallas_call(
                _tc_body_acc,
                out_shape=jax.ShapeDtypeStruct((L, B, D), jnp.float32),
                grid=(B // BBC,),
                in_specs=[pl.BlockSpec(memory_space=pl.ANY)] + data_specs,
                out_specs=out_spec,
                input_output_aliases={0: 0},
            )(outT, *args)
    return jnp.transpose(outT, (1, 0, 2))
